# baseline (speedup 1.0000x reference)
import functools

import jax
import jax.numpy as jnp
from jax import lax
from jax.experimental import pallas as pl
from jax.experimental.pallas import tpu as pltpu
from jax.experimental.pallas import tpu_sc as plsc

B, N, C = 16, 8192, 256
LEVEL_SIZES = (56, 28, 14, 7)
NLVL = 4
FX, FY, CX, CY = 250.0, 250.0, 112.0, 112.0
IMG_H, IMG_W = 224.0, 224.0
OUTD = 3 + NLVL * C
PTS = B * N

NC, NS, L = 2, 16, 16
NW = NC * NS
PPW = PTS // NW
CHUNK = 16
NCHUNK = PPW // CHUNK
OUTB = CHUNK * OUTD
NROW = 4 * L


def _sc_relayout(img_feats):
    mesh = plsc.VectorSubcoreMesh(
        core_axis_name="c", subcore_axis_name="s",
        num_cores=NC, num_subcores=NS)

    HW = LEVEL_SIZES[0]
    scratch = [pltpu.VMEM((C, HW), jnp.float32),
               pltpu.VMEM((HW, 2, 128), jnp.bfloat16)]

    @functools.partial(
        pl.kernel,
        out_type=tuple(
            jax.ShapeDtypeStruct((B * s_ * s_, 2, 128), jnp.bfloat16)
            for s_ in LEVEL_SIZES),
        mesh=mesh,
        compiler_params=pltpu.CompilerParams(
            needs_layout_passes=False, use_tc_tiling_on_sc=False),
        scratch_types=scratch,
    )
    def k(feats_hbm, o0, o1, o2, o3, blk, stg):
        outs = (o0, o1, o2, o3)
        wid = lax.axis_index("s") * NC + lax.axis_index("c")
        lane = lax.iota(jnp.int32, L)
        ilv = plsc.PackFormat.INTERLEAVED

        for lvl, size in enumerate(LEVEL_SIZES):
            nx = B * size
            per = -(-nx // NW)
            out = outs[lvl]

            def xbody(i, _, lvl=lvl, size=size, nx=nx, per=per, out=out):
                xrow = wid * per + i

                @pl.when(xrow < nx)
                def _():
                    bi = xrow // size
                    x = lax.rem(xrow, size)
                    pltpu.sync_copy(feats_hbm.at[lvl, bi, :, x], blk)

                    @plsc.parallel_loop(0, size, unroll=2)
                    def ybody(y):
                        yfull = jnp.full((L,), y, jnp.int32)
                        for g in range(C // 32):
                            a_lo = plsc.load_gather(
                                blk, [g * 32 + lane, yfull])
                            a_hi = plsc.load_gather(
                                blk, [g * 32 + 16 + lane, yfull])
                            rif = plsc.pack(a_lo, a_hi, format=ilv)
                            stg[y, g // 4, pl.ds((g % 4) * 32, 32)] = rif

                    pltpu.sync_copy(
                        stg.at[pl.ds(0, size)],
                        out.at[pl.ds(bi * size * size + x * size, size)])
                return 0

            lax.fori_loop(0, per, xbody, 0)

    return k(img_feats)


def _sc_project(verts_t, tables):
    mesh = plsc.VectorSubcoreMesh(
        core_axis_name="c", subcore_axis_name="s",
        num_cores=NC, num_subcores=NS)

    @functools.partial(
        pl.kernel,
        out_type=jax.ShapeDtypeStruct((B, N, OUTD), jnp.float32),
        mesh=mesh,
        compiler_params=pltpu.CompilerParams(
            needs_layout_passes=False, use_tc_tiling_on_sc=False),
        scratch_types=[
            pltpu.VMEM((3, PPW), jnp.float32),
            pltpu.VMEM((2, NLVL, NROW), jnp.int32),
            pltpu.VMEM((2, NLVL, NROW, 2, 128), jnp.bfloat16),
            pltpu.VMEM((2, CHUNK, OUTD), jnp.float32),
            pltpu.SemaphoreType.DMA,
            pltpu.SemaphoreType.DMA,
            pltpu.SemaphoreType.DMA,
        ],
    )
    def k(verts_hbm, tbl0, tbl1, tbl2, tbl3, out_hbm, verts_v, idx_v,
          rows_v, outb_v, sem_g0, sem_g1, sem_out):
        tbls = (tbl0, tbl1, tbl2, tbl3)
        wid = lax.axis_index("s") * NC + lax.axis_index("c")
        base = wid * PPW
        bidx = base // N
        nbase = lax.rem(base, N)
        pltpu.sync_copy(verts_hbm.at[:, pl.ds(base, PPW)], verts_v)

        lane = lax.iota(jnp.int32, L)
        sems = (sem_g0, sem_g1)
        ilv = plsc.PackFormat.INTERLEAVED

        def coords(off):
            xv = verts_v[0, pl.ds(off, L)]
            yv = verts_v[1, pl.ds(off, L)]
            zv = verts_v[2, pl.ds(off, L)]
            h = FY * (yv / zv) + CY
            w = FX * (xv / (-zv)) + CX
            return xv, yv, zv, h, w

        def level_coords(h, w, size):
            x = jnp.clip(h * (size / IMG_H), 0.0, size - 1.0)
            y = jnp.clip(w * (size / IMG_W), 0.0, size - 1.0)
            x1i = x.astype(jnp.int32)
            x1f = x1i.astype(jnp.float32)
            x2i = x1i + (x > x1f).astype(jnp.int32)
            x2f = x2i.astype(jnp.float32)
            y1i = y.astype(jnp.int32)
            y1f = y1i.astype(jnp.float32)
            y2i = y1i + (y > y1f).astype(jnp.int32)
            y2f = y2i.astype(jnp.float32)
            return x, y, x1i, x1f, x2i, x2f, y1i, y1f, y2i, y2f

        def write_and_issue(sel, off):
            _, _, _, h, w = coords(off)
            for lvl, size in enumerate(LEVEL_SIZES):
                _, _, x1i, _, x2i, _, y1i, _, y2i, _ = level_coords(h, w, size)
                rowbase = bidx * (size * size)
                r1 = rowbase + x1i * size
                r2 = rowbase + x2i * size
                idx_v[sel, lvl, pl.ds(0 * L, L)] = r1 + y1i
                idx_v[sel, lvl, pl.ds(1 * L, L)] = r2 + y1i
                idx_v[sel, lvl, pl.ds(2 * L, L)] = r1 + y2i
                idx_v[sel, lvl, pl.ds(3 * L, L)] = r2 + y2i
            for lvl in range(NLVL):
                pltpu.async_copy(tbls[lvl].at[idx_v.at[sel, lvl]],
                                 rows_v.at[sel, lvl], sems[sel])

        def do_chunk(ci, par, off):
            nxt = 1 - par

            if par == 0:
                write_and_issue(nxt, off + CHUNK)
            else:
                @pl.when(ci + 1 < NCHUNK)
                def _():
                    write_and_issue(nxt, off + CHUNK)

            for lvl in range(NLVL):
                pltpu.make_async_copy(tbls[lvl].at[idx_v.at[par, lvl]],
                                      rows_v.at[par, lvl], sems[par]).wait()

            xv, yv, zv, h, w = coords(off)

            parf = jnp.full((L,), par, jnp.int32)
            plsc.store_scatter(outb_v, [parf, lane, jnp.full((L,), 0, jnp.int32)], xv)
            plsc.store_scatter(outb_v, [parf, lane, jnp.full((L,), 1, jnp.int32)], yv)
            plsc.store_scatter(outb_v, [parf, lane, jnp.full((L,), 2, jnp.int32)], zv)

            wts = []
            for lvl, size in enumerate(LEVEL_SIZES):
                x, y, _, x1f, _, x2f, _, y1f, _, y2f = level_coords(h, w, size)
                wts.append(((x2f - x) * (y2f - y), (x - x1f) * (y2f - y),
                            (x2f - x) * (y - y1f), (x - x1f) * (y - y1f)))

            @plsc.parallel_loop(0, CHUNK, unroll=8)
            def blend_p(p):
                pfull = jnp.full((L,), p, jnp.int32)
                for lvl in range(NLVL):
                    w11v, w21v, w12v, w22v = wts[lvl]
                    w11 = w11v.at[pfull].get(mode="promise_in_bounds")
                    w21 = w21v.at[pfull].get(mode="promise_in_bounds")
                    w12 = w12v.at[pfull].get(mode="promise_in_bounds")
                    w22 = w22v.at[pfull].get(mode="promise_in_bounds")
                    wb11 = plsc.pack(w11, w11, format=ilv)
                    wb21 = plsc.pack(w21, w21, format=ilv)
                    wb12 = plsc.pack(w12, w12, format=ilv)
                    wb22 = plsc.pack(w22, w22, format=ilv)
                    dst0 = 3 + lvl * C
                    for g in range(C // (2 * L)):
                        half, sl = g // 4, pl.ds((g % 4) * 2 * L, 2 * L)
                        acc = (wb11 * rows_v[par, lvl, p, half, sl]
                               + wb21 * rows_v[par, lvl, L + p, half, sl]
                               + wb12 * rows_v[par, lvl, 2 * L + p, half, sl]
                               + wb22 * rows_v[par, lvl, 3 * L + p, half, sl])
                        lo, hi = plsc.unpack(acc, format=ilv)
                        outb_v[par, p, pl.ds(dst0 + g * 2 * L, L)] = lo
                        outb_v[par, p, pl.ds(dst0 + g * 2 * L + L, L)] = hi

            def drain_prev():
                pltpu.make_async_copy(
                    outb_v.at[nxt],
                    out_hbm.at[bidx, pl.ds(nbase + off - CHUNK, CHUNK), :],
                    sem_out).wait()

            if par == 1:
                drain_prev()
            else:
                @pl.when(ci > 0)
                def _():
                    drain_prev()

            pltpu.async_copy(outb_v.at[par],
                             out_hbm.at[bidx, pl.ds(nbase + off, CHUNK), :],
                             sem_out)

        write_and_issue(0, 0)

        def pair_body(cj, _):
            ci = 2 * cj
            off = ci * CHUNK
            do_chunk(ci, 0, off)
            do_chunk(ci + 1, 1, off + CHUNK)
            return 0

        lax.fori_loop(0, NCHUNK // 2, pair_body, 0)

        last_off = (NCHUNK - 1) * CHUNK
        pltpu.make_async_copy(
            outb_v.at[(NCHUNK - 1) % 2],
            out_hbm.at[bidx, pl.ds(nbase + last_off, CHUNK), :],
            sem_out).wait()

    return k(verts_t, *tables)


def kernel(vertices, img_feats, proj_mat):
    del proj_mat
    tables = _sc_relayout(img_feats)
    verts_t = jnp.transpose(vertices.reshape(PTS, 3), (1, 0))
    return _sc_project(verts_t, tables)


# speedup vs baseline: 1.2372x; 1.2372x over previous
import functools

import jax
import jax.numpy as jnp
from jax import lax
from jax.experimental import pallas as pl
from jax.experimental.pallas import tpu as pltpu
from jax.experimental.pallas import tpu_sc as plsc

B, N, C = 16, 8192, 256
LEVEL_SIZES = (56, 28, 14, 7)
NLVL = 4
FX, FY, CX, CY = 250.0, 250.0, 112.0, 112.0
IMG_H, IMG_W = 224.0, 224.0
OUTD = 3 + NLVL * C
PTS = B * N

NC, NS, L = 2, 16, 16
NW = NC * NS
PPW = PTS // NW
CHUNK = 16
NCHUNK = PPW // CHUNK
OUTB = CHUNK * OUTD
NROW = 4 * L


def _sc_relayout(feats_t):
    mesh = plsc.VectorSubcoreMesh(
        core_axis_name="c", subcore_axis_name="s",
        num_cores=NC, num_subcores=NS)

    HW = LEVEL_SIZES[0]
    scratch = [pltpu.VMEM((HW, C), jnp.float32),
               pltpu.VMEM((HW, 2, 128), jnp.bfloat16)]

    @functools.partial(
        pl.kernel,
        out_type=tuple(
            jax.ShapeDtypeStruct((B * s_ * s_, 2, 128), jnp.bfloat16)
            for s_ in LEVEL_SIZES),
        mesh=mesh,
        compiler_params=pltpu.CompilerParams(
            needs_layout_passes=False, use_tc_tiling_on_sc=False),
        scratch_types=scratch,
    )
    def k(feats_hbm, o0, o1, o2, o3, blk, stg):
        outs = (o0, o1, o2, o3)
        wid = lax.axis_index("s") * NC + lax.axis_index("c")
        lane = lax.iota(jnp.int32, L)
        ilv = plsc.PackFormat.INTERLEAVED

        for lvl, size in enumerate(LEVEL_SIZES):
            nx = B * size
            per = -(-nx // NW)
            out = outs[lvl]

            def xbody(i, _, lvl=lvl, size=size, nx=nx, per=per, out=out):
                xrow = wid * per + i

                @pl.when(xrow < nx)
                def _():
                    bi = xrow // size
                    x = lax.rem(xrow, size)
                    pltpu.sync_copy(feats_hbm.at[lvl, bi, x], blk)

                    @plsc.parallel_loop(0, size, unroll=2)
                    def ybody(y):
                        for g in range(C // 32):
                            a_lo = blk[y, pl.ds(g * 32, L)]
                            a_hi = blk[y, pl.ds(g * 32 + L, L)]
                            rif = plsc.pack(a_lo, a_hi, format=ilv)
                            stg[y, g // 4, pl.ds((g % 4) * 32, 32)] = rif

                    pltpu.sync_copy(
                        stg.at[pl.ds(0, size)],
                        out.at[pl.ds(bi * size * size + x * size, size)])
                return 0

            lax.fori_loop(0, per, xbody, 0)

    return k(feats_t)


def _sc_project(verts_t, tables):
    mesh = plsc.VectorSubcoreMesh(
        core_axis_name="c", subcore_axis_name="s",
        num_cores=NC, num_subcores=NS)

    @functools.partial(
        pl.kernel,
        out_type=jax.ShapeDtypeStruct((B, N, OUTD), jnp.float32),
        mesh=mesh,
        compiler_params=pltpu.CompilerParams(
            needs_layout_passes=False, use_tc_tiling_on_sc=False),
        scratch_types=[
            pltpu.VMEM((3, PPW), jnp.float32),
            pltpu.VMEM((2, NLVL, NROW), jnp.int32),
            pltpu.VMEM((2, NLVL, NROW, 2, 128), jnp.bfloat16),
            pltpu.VMEM((2, CHUNK, OUTD), jnp.float32),
            pltpu.SemaphoreType.DMA,
            pltpu.SemaphoreType.DMA,
            pltpu.SemaphoreType.DMA,
        ],
    )
    def k(verts_hbm, tbl0, tbl1, tbl2, tbl3, out_hbm, verts_v, idx_v,
          rows_v, outb_v, sem_g0, sem_g1, sem_out):
        tbls = (tbl0, tbl1, tbl2, tbl3)
        wid = lax.axis_index("s") * NC + lax.axis_index("c")
        base = wid * PPW
        bidx = base // N
        nbase = lax.rem(base, N)
        pltpu.sync_copy(verts_hbm.at[:, pl.ds(base, PPW)], verts_v)

        lane = lax.iota(jnp.int32, L)
        sems = (sem_g0, sem_g1)
        ilv = plsc.PackFormat.INTERLEAVED

        def coords(off):
            xv = verts_v[0, pl.ds(off, L)]
            yv = verts_v[1, pl.ds(off, L)]
            zv = verts_v[2, pl.ds(off, L)]
            h = FY * (yv / zv) + CY
            w = FX * (xv / (-zv)) + CX
            return xv, yv, zv, h, w

        def level_coords(h, w, size):
            x = jnp.clip(h * (size / IMG_H), 0.0, size - 1.0)
            y = jnp.clip(w * (size / IMG_W), 0.0, size - 1.0)
            x1i = x.astype(jnp.int32)
            x1f = x1i.astype(jnp.float32)
            x2i = x1i + (x > x1f).astype(jnp.int32)
            x2f = x2i.astype(jnp.float32)
            y1i = y.astype(jnp.int32)
            y1f = y1i.astype(jnp.float32)
            y2i = y1i + (y > y1f).astype(jnp.int32)
            y2f = y2i.astype(jnp.float32)
            return x, y, x1i, x1f, x2i, x2f, y1i, y1f, y2i, y2f

        def write_and_issue(sel, off):
            _, _, _, h, w = coords(off)
            for lvl, size in enumerate(LEVEL_SIZES):
                _, _, x1i, _, x2i, _, y1i, _, y2i, _ = level_coords(h, w, size)
                rowbase = bidx * (size * size)
                r1 = rowbase + x1i * size
                r2 = rowbase + x2i * size
                idx_v[sel, lvl, pl.ds(0 * L, L)] = r1 + y1i
                idx_v[sel, lvl, pl.ds(1 * L, L)] = r2 + y1i
                idx_v[sel, lvl, pl.ds(2 * L, L)] = r1 + y2i
                idx_v[sel, lvl, pl.ds(3 * L, L)] = r2 + y2i
            for lvl in range(NLVL):
                pltpu.async_copy(tbls[lvl].at[idx_v.at[sel, lvl]],
                                 rows_v.at[sel, lvl], sems[sel])

        def do_chunk(ci, par, off):
            nxt = 1 - par

            if par == 0:
                write_and_issue(nxt, off + CHUNK)
            else:
                @pl.when(ci + 1 < NCHUNK)
                def _():
                    write_and_issue(nxt, off + CHUNK)

            for lvl in range(NLVL):
                pltpu.make_async_copy(tbls[lvl].at[idx_v.at[par, lvl]],
                                      rows_v.at[par, lvl], sems[par]).wait()

            xv, yv, zv, h, w = coords(off)

            parf = jnp.full((L,), par, jnp.int32)
            plsc.store_scatter(outb_v, [parf, lane, jnp.full((L,), 0, jnp.int32)], xv)
            plsc.store_scatter(outb_v, [parf, lane, jnp.full((L,), 1, jnp.int32)], yv)
            plsc.store_scatter(outb_v, [parf, lane, jnp.full((L,), 2, jnp.int32)], zv)

            wts = []
            for lvl, size in enumerate(LEVEL_SIZES):
                x, y, _, x1f, _, x2f, _, y1f, _, y2f = level_coords(h, w, size)
                wts.append(((x2f - x) * (y2f - y), (x - x1f) * (y2f - y),
                            (x2f - x) * (y - y1f), (x - x1f) * (y - y1f)))

            @plsc.parallel_loop(0, CHUNK, unroll=4)
            def blend_p(p):
                pfull = jnp.full((L,), p, jnp.int32)
                for lvl in range(NLVL):
                    w11v, w21v, w12v, w22v = wts[lvl]
                    w11 = w11v.at[pfull].get(mode="promise_in_bounds")
                    w21 = w21v.at[pfull].get(mode="promise_in_bounds")
                    w12 = w12v.at[pfull].get(mode="promise_in_bounds")
                    w22 = w22v.at[pfull].get(mode="promise_in_bounds")
                    wb11 = plsc.pack(w11, w11, format=ilv)
                    wb21 = plsc.pack(w21, w21, format=ilv)
                    wb12 = plsc.pack(w12, w12, format=ilv)
                    wb22 = plsc.pack(w22, w22, format=ilv)
                    dst0 = 3 + lvl * C
                    for g in range(C // (2 * L)):
                        half, sl = g // 4, pl.ds((g % 4) * 2 * L, 2 * L)
                        acc = (wb11 * rows_v[par, lvl, p, half, sl]
                               + wb21 * rows_v[par, lvl, L + p, half, sl]
                               + wb12 * rows_v[par, lvl, 2 * L + p, half, sl]
                               + wb22 * rows_v[par, lvl, 3 * L + p, half, sl])
                        lo, hi = plsc.unpack(acc, format=ilv)
                        outb_v[par, p, pl.ds(dst0 + g * 2 * L, L)] = lo
                        outb_v[par, p, pl.ds(dst0 + g * 2 * L + L, L)] = hi

            def drain_prev():
                pltpu.make_async_copy(
                    outb_v.at[nxt],
                    out_hbm.at[bidx, pl.ds(nbase + off - CHUNK, CHUNK), :],
                    sem_out).wait()

            if par == 1:
                drain_prev()
            else:
                @pl.when(ci > 0)
                def _():
                    drain_prev()

            pltpu.async_copy(outb_v.at[par],
                             out_hbm.at[bidx, pl.ds(nbase + off, CHUNK), :],
                             sem_out)

        write_and_issue(0, 0)

        def pair_body(cj, _):
            ci = 2 * cj
            off = ci * CHUNK
            do_chunk(ci, 0, off)
            do_chunk(ci + 1, 1, off + CHUNK)
            return 0

        lax.fori_loop(0, NCHUNK // 2, pair_body, 0)

        last_off = (NCHUNK - 1) * CHUNK
        pltpu.make_async_copy(
            outb_v.at[(NCHUNK - 1) % 2],
            out_hbm.at[bidx, pl.ds(nbase + last_off, CHUNK), :],
            sem_out).wait()

    return k(verts_t, *tables)


def kernel(vertices, img_feats, proj_mat):
    del proj_mat
    tables = _sc_relayout(jnp.transpose(img_feats, (0, 1, 3, 4, 2)))
    verts_t = jnp.transpose(vertices.reshape(PTS, 3), (1, 0))
    return _sc_project(verts_t, tables)
